# baseline (speedup 1.0000x reference)
import functools

import jax
import jax.numpy as jnp
from jax import lax
from jax.experimental import pallas as pl
from jax.experimental.pallas import tpu as pltpu
from jax.experimental.pallas import tpu_sc as plsc

NUM_EXPERTS = 8
TOP_K = 2
D_MODEL = 1024
HIDDEN = 2048
TOKENS = 8192

BLK = 256
FLAT = TOKENS * TOP_K
NT = FLAT // BLK + NUM_EXPERTS
PTOT = NT * BLK

NC, NS = 2, 16
NW = NC * NS
TPW = TOKENS // NW
CHUNK = 64
IW = D_MODEL // 2 // 128


def _dispatch_body(xb, i0, i1, out, rows_v, idx0_v, idx1_v, sem):
    wid = lax.axis_index("s") * NC + lax.axis_index("c")
    base = wid * TPW
    for it in range(TPW // CHUNK):
        tok = base + it * CHUNK
        pltpu.sync_copy(xb.at[pl.ds(tok, CHUNK)], rows_v)
        pltpu.sync_copy(i0.at[pl.ds(tok, CHUNK)], idx0_v)
        pltpu.sync_copy(i1.at[pl.ds(tok, CHUNK)], idx1_v)
        pltpu.async_copy(rows_v, out.at[idx0_v], sem).wait()
        pltpu.async_copy(rows_v, out.at[idx1_v], sem).wait()


_dispatch = functools.partial(
    pl.kernel,
    mesh=plsc.VectorSubcoreMesh(core_axis_name="c", subcore_axis_name="s"),
    out_type=jax.ShapeDtypeStruct((PTOT, IW, 128), jnp.int32),
    scratch_types=[
        pltpu.VMEM((CHUNK, IW, 128), jnp.int32),
        pltpu.VMEM((CHUNK,), jnp.int32),
        pltpu.VMEM((CHUNK,), jnp.int32),
        pltpu.SemaphoreType.DMA,
    ],
)(_dispatch_body)


def _mlp_body(e_ref, xg_ref, w1_ref, b1_ref, w2_ref, b2_ref, o_ref):
    xb = xg_ref[...]
    w1 = w1_ref[0]
    h = lax.dot_general(xb, w1, (((1,), (1,)), ((), ())),
                        preferred_element_type=jnp.float32)
    h = jax.nn.relu(h + b1_ref[0])
    w2 = w2_ref[0]
    o = lax.dot_general(h.astype(jnp.bfloat16), w2, (((1,), (1,)), ((), ())),
                        preferred_element_type=jnp.float32)
    o_ref[...] = jax.nn.sigmoid(o + b2_ref[0])


def _grouped_mlp(tile_expert, xg, W1, b1, W2, b2):
    grid_spec = pltpu.PrefetchScalarGridSpec(
        num_scalar_prefetch=1,
        grid=(NT,),
        in_specs=[
            pl.BlockSpec((BLK, D_MODEL), lambda i, e: (i, 0)),
            pl.BlockSpec((1, HIDDEN, D_MODEL), lambda i, e: (e[i], 0, 0)),
            pl.BlockSpec((1, 1, HIDDEN), lambda i, e: (e[i], 0, 0)),
            pl.BlockSpec((1, D_MODEL, HIDDEN), lambda i, e: (e[i], 0, 0)),
            pl.BlockSpec((1, 1, D_MODEL), lambda i, e: (e[i], 0, 0)),
        ],
        out_specs=pl.BlockSpec((BLK, D_MODEL), lambda i, e: (i, 0)),
    )
    return pl.pallas_call(
        _mlp_body,
        grid_spec=grid_spec,
        out_shape=jax.ShapeDtypeStruct((PTOT, D_MODEL), jnp.float32),
    )(tile_expert, xg, W1, b1.reshape(NUM_EXPERTS, 1, HIDDEN), W2,
      b2.reshape(NUM_EXPERTS, 1, D_MODEL))


def kernel(x, Wr, br, W1, b1, W2, b2):
    T = TOKENS
    logits = x @ Wr.T + br
    top_v, top_i = lax.top_k(logits, TOP_K)
    top_w = jax.nn.softmax(top_v, axis=-1)

    e_flat = top_i.reshape(-1).astype(jnp.int32)
    onehot = (e_flat[:, None] == jnp.arange(NUM_EXPERTS, dtype=jnp.int32)[None, :]
              ).astype(jnp.int32)
    incl = jnp.cumsum(onehot, axis=0)
    counts = incl[-1]
    rank = jnp.take_along_axis(incl - onehot, e_flat[:, None], axis=1)[:, 0]
    padded = ((counts + BLK - 1) // BLK) * BLK
    offs = jnp.concatenate([jnp.zeros((1,), jnp.int32),
                            jnp.cumsum(padded)[:-1].astype(jnp.int32)])
    slot = offs[e_flat] + rank
    inv = slot.reshape(T, TOP_K)
    idx0 = inv[:, 0].astype(jnp.int32)
    idx1 = inv[:, 1].astype(jnp.int32)

    cum_end = jnp.cumsum(padded)
    tile_start = jnp.arange(NT, dtype=jnp.int32) * BLK
    tile_expert = jnp.sum(
        (tile_start[:, None] >= cum_end[None, :]).astype(jnp.int32), axis=1)
    tile_expert = jnp.minimum(tile_expert, NUM_EXPERTS - 1).astype(jnp.int32)

    xb3 = lax.bitcast_convert_type(
        x.astype(jnp.bfloat16).reshape(T, D_MODEL // 2, 2), jnp.int32
    ).reshape(T, IW, 128)
    xg = lax.bitcast_convert_type(
        _dispatch(xb3, idx0, idx1).reshape(PTOT, D_MODEL // 2),
        jnp.bfloat16).reshape(PTOT, D_MODEL)
    o_buf = _grouped_mlp(tile_expert, xg, W1.astype(jnp.bfloat16), b1,
                         W2.astype(jnp.bfloat16), b2)
    out = (top_w[:, 0:1] * jnp.take(o_buf, idx0, axis=0)
           + top_w[:, 1:2] * jnp.take(o_buf, idx1, axis=0))
    return out
